# Initial kernel scaffold; baseline (speedup 1.0000x reference)
import jax
import jax.numpy as jnp
from jax.experimental import pallas as pl


def kernel(x, edge_index, W1, a_src1, a_dst1, b1, W2, a_src2, a_dst2, b2):
    raise NotImplementedError("write your pallas kernel here")


# speedup vs baseline: 18.4381x; 18.4381x over previous
import dataclasses
import functools

import jax
import jax.numpy as jnp
from jax import lax
from jax.experimental import pallas as pl
from jax.experimental.pallas import tpu as pltpu
from jax.experimental.pallas import tpu_sc as plsc

N = 10000
D = 128
C = 128
E = 320000

NC = 2
NS = 16
NW = NC * NS
L = 16

CHUNK = 128
NCHUNK = 79
EPT = NCHUNK * CHUNK
E_PAD = NW * EPT
N_ACC = 10240
ZPT = N_ACC // NS
A_PAD = 10016

BLK = 400
GRID = N // BLK


def _sc_compiler_params():
    cp = pltpu.CompilerParams()
    if "needs_layout_passes" in pltpu.CompilerParams.__dataclass_fields__:
        cp = dataclasses.replace(cp, needs_layout_passes=False)
    return cp


def _proj_body(x_ref, w_ref, asr_ref, adr_ref, h_ref, a1_ref, a2_ref):
    h = jnp.dot(x_ref[...], w_ref[...], preferred_element_type=jnp.float32)
    h_ref[...] = h
    a1_ref[...] = jnp.sum(h * asr_ref[...], axis=1, keepdims=True)
    a2_ref[...] = jnp.sum(h * adr_ref[...], axis=1, keepdims=True)


def _proj(x, w, a_src, a_dst):
    return pl.pallas_call(
        _proj_body,
        grid=(GRID,),
        in_specs=[
            pl.BlockSpec((BLK, D), lambda i: (i, 0)),
            pl.BlockSpec((D, C), lambda i: (0, 0)),
            pl.BlockSpec((1, C), lambda i: (0, 0)),
            pl.BlockSpec((1, C), lambda i: (0, 0)),
        ],
        out_specs=[
            pl.BlockSpec((BLK, C), lambda i: (i, 0)),
            pl.BlockSpec((BLK, 1), lambda i: (i, 0)),
            pl.BlockSpec((BLK, 1), lambda i: (i, 0)),
        ],
        out_shape=[
            jax.ShapeDtypeStruct((N, C), jnp.float32),
            jax.ShapeDtypeStruct((N, 1), jnp.float32),
            jax.ShapeDtypeStruct((N, 1), jnp.float32),
        ],
    )(x, w, a_src, a_dst)


def _finish(acc_ref, den_ref, b_ref):
    acc = acc_ref[0] + acc_ref[1]
    den = den_ref[0] + den_ref[1]
    o = acc / (den + 1e-16) + b_ref[...]
    return jnp.where(o > 0.0, o, jnp.exp(o) - 1.0)


def _finproj_body(acc_ref, den_ref, b_ref, w_ref, asr_ref, adr_ref,
                  h_ref, a1_ref, a2_ref):
    hin = _finish(acc_ref, den_ref, b_ref)
    h = jnp.dot(hin, w_ref[...], preferred_element_type=jnp.float32)
    h_ref[...] = h
    a1_ref[...] = jnp.sum(h * asr_ref[...], axis=1, keepdims=True)
    a2_ref[...] = jnp.sum(h * adr_ref[...], axis=1, keepdims=True)


def _finproj(acc, den, b, w, a_src, a_dst):
    return pl.pallas_call(
        _finproj_body,
        grid=(GRID,),
        in_specs=[
            pl.BlockSpec((NC, BLK, C), lambda i: (0, i, 0)),
            pl.BlockSpec((NC, BLK, 1), lambda i: (0, i, 0)),
            pl.BlockSpec((1, C), lambda i: (0, 0)),
            pl.BlockSpec((D, C), lambda i: (0, 0)),
            pl.BlockSpec((1, C), lambda i: (0, 0)),
            pl.BlockSpec((1, C), lambda i: (0, 0)),
        ],
        out_specs=[
            pl.BlockSpec((BLK, C), lambda i: (i, 0)),
            pl.BlockSpec((BLK, 1), lambda i: (i, 0)),
            pl.BlockSpec((BLK, 1), lambda i: (i, 0)),
        ],
        out_shape=[
            jax.ShapeDtypeStruct((N, C), jnp.float32),
            jax.ShapeDtypeStruct((N, 1), jnp.float32),
            jax.ShapeDtypeStruct((N, 1), jnp.float32),
        ],
    )(acc, den, b, w, a_src, a_dst)


def _final_body(acc_ref, den_ref, b_ref, o_ref):
    o_ref[...] = _finish(acc_ref, den_ref, b_ref)


def _final(acc, den, b):
    return pl.pallas_call(
        _final_body,
        grid=(GRID,),
        in_specs=[
            pl.BlockSpec((NC, BLK, C), lambda i: (0, i, 0)),
            pl.BlockSpec((NC, BLK, 1), lambda i: (0, i, 0)),
            pl.BlockSpec((1, C), lambda i: (0, 0)),
        ],
        out_specs=pl.BlockSpec((BLK, C), lambda i: (i, 0)),
        out_shape=jax.ShapeDtypeStruct((N, C), jnp.float32),
    )(acc, den, b)


_MESH = plsc.VectorSubcoreMesh(core_axis_name="core", subcore_axis_name="subcore")


@functools.partial(
    pl.kernel,
    out_type=[
        jax.ShapeDtypeStruct((NC * N_ACC, C), jnp.float32),
        jax.ShapeDtypeStruct((NC * N_ACC,), jnp.float32),
    ],
    mesh=_MESH,
    compiler_params=_sc_compiler_params(),
    scratch_types=[
        pltpu.VMEM((NCHUNK, CHUNK), jnp.int32),
        pltpu.VMEM((NCHUNK, CHUNK), jnp.int32),
        pltpu.VMEM((CHUNK,), jnp.float32),
        pltpu.VMEM((CHUNK,), jnp.float32),
        pltpu.VMEM((CHUNK,), jnp.float32),
        pltpu.VMEM((CHUNK, C), jnp.float32),
        pltpu.VMEM_SHARED((N_ACC, C), jnp.float32),
        pltpu.VMEM_SHARED((N_ACC,), jnp.float32),
        pltpu.SemaphoreType.DMA,
        pltpu.SemaphoreType.DMA,
    ],
)
def _sc_aggregate(h_hbm, asrc_hbm, adst_hbm, sidx_hbm, didx_hbm,
                  zrows_hbm, zden_hbm, acc_out, den_out,
                  sidx_v, didx_v, asv, adv, wbuf, rows_v,
                  acc_s, den_s, sem, rsem):
    cid = lax.axis_index("core")
    sid = lax.axis_index("subcore")
    wid = sid * NC + cid
    zbase = sid * ZPT

    pltpu.sync_copy(zrows_hbm, acc_s.at[pl.ds(zbase, ZPT)])
    pltpu.sync_copy(zden_hbm, den_s.at[pl.ds(zbase, ZPT)])

    pltpu.sync_copy(sidx_hbm.at[wid], sidx_v)
    pltpu.sync_copy(didx_hbm.at[wid], didx_v)

    plsc.subcore_barrier()

    @pl.loop(0, NCHUNK)
    def _p2(j):
        rows_cp = pltpu.async_copy(h_hbm.at[sidx_v.at[j]], rows_v, rsem)
        pltpu.async_copy(asrc_hbm.at[sidx_v.at[j]], asv, sem).wait()
        pltpu.async_copy(adst_hbm.at[didx_v.at[j]], adv, sem).wait()

        for k in range(CHUNK // L):
            sl = pl.ds(L * k, L)
            e = asv.at[sl][...] + adv.at[sl][...]
            e = jnp.where(e > 0.0, e, 0.2 * e)
            wbuf.at[sl][...] = jnp.exp(e)

        pltpu.sync_copy(wbuf, den_s.at[didx_v.at[j]], add=True)
        rows_cp.wait()

        @pl.loop(0, CHUNK)
        def _scale(e):
            ee = jnp.full((L,), e, jnp.int32)
            wsp = plsc.load_gather(wbuf, [ee])
            for b in range(C // L):
                sl = pl.ds(L * b, L)
                rows_v.at[e, sl][...] = rows_v.at[e, sl][...] * wsp

        pltpu.sync_copy(rows_v, acc_s.at[didx_v.at[j]], add=True)

    plsc.subcore_barrier()

    obase = cid * N_ACC + zbase
    pltpu.sync_copy(acc_s.at[pl.ds(zbase, ZPT)], acc_out.at[pl.ds(obase, ZPT)])
    pltpu.sync_copy(den_s.at[pl.ds(zbase, ZPT)], den_out.at[pl.ds(obase, ZPT)])


def _layer_aggregate(h, asrc, adst, sidx3, didx3, zrows, zden):
    asrc_p = jnp.pad(asrc[:, 0], (0, A_PAD - N))
    adst_p = jnp.pad(adst[:, 0], (0, A_PAD - N))
    acc, den = _sc_aggregate(h, asrc_p, adst_p, sidx3, didx3, zrows, zden)
    acc = acc.reshape(NC, N_ACC, C)
    den = den.reshape(NC, N_ACC, 1)
    return acc, den


def kernel(x, edge_index, W1, a_src1, a_dst1, b1, W2, a_src2, a_dst2, b2):
    src = edge_index[0]
    dst = edge_index[1]
    src_p = jnp.concatenate([src, jnp.zeros((E_PAD - E,), jnp.int32)])
    dst_p = jnp.concatenate([dst, jnp.full((E_PAD - E,), N, jnp.int32)])
    sidx3 = src_p.reshape(NW, NCHUNK, CHUNK)
    didx3 = dst_p.reshape(NW, NCHUNK, CHUNK)
    zrows = jnp.zeros((ZPT, C), jnp.float32)
    zden = jnp.zeros((ZPT,), jnp.float32)

    b1r = b1.reshape(1, C)
    b2r = b2.reshape(1, C)

    h1, as1, ad1 = _proj(x, W1, a_src1, a_dst1)
    acc1, den1 = _layer_aggregate(h1, as1, ad1, sidx3, didx3, zrows, zden)
    h2, as2, ad2 = _finproj(acc1, den1, b1r, W2, a_src2, a_dst2)
    acc2, den2 = _layer_aggregate(h2, as2, ad2, sidx3, didx3, zrows, zden)
    return _final(acc2, den2, b2r)
